# baseline (speedup 1.0000x reference)
import functools

import jax
import jax.numpy as jnp
import numpy as np
from jax import lax
from jax.experimental import pallas as pl
from jax.experimental.pallas import tpu as pltpu
from jax.experimental.pallas import tpu_sc as plsc

N_NODES = 10000
N_EDGES = 160000
D = 256

NC = 2
NS = 16
NW = NC * NS
ROWS = 320
TRASH = ROWS
SLAB = ROWS + 8
SENT = N_NODES
SCE = 1600
NSC = N_EDGES // SCE
GB = 64
LOGCAP = SCE + 32
BIG = np.int32(1 << 20)

_GDN = lax.GatherDimensionNumbers(offset_dims=(), collapsed_slice_dims=(0,),
                                  start_index_map=(0,))


def _take16(v, idx):
    return lax.gather(v, idx.reshape(16, 1), _GDN, (1,),
                      mode=lax.GatherScatterMode.PROMISE_IN_BOUNDS)


def _bmin(v, perms):
    for p in perms:
        v = jnp.minimum(v, _take16(v, p))
    return v


def _bsum(v, perms):
    for p in perms:
        v = v + _take16(v, p)
    return v


def _sc_agg_build():
    mesh = plsc.VectorSubcoreMesh(core_axis_name="c", subcore_axis_name="s")

    @functools.partial(
        pl.kernel,
        mesh=mesh,
        out_type=jax.ShapeDtypeStruct((NW * ROWS, D), jnp.float32),
        scratch_types=[
            pltpu.VMEM((2 * SCE,), jnp.int32),
            pltpu.VMEM((2 * SCE,), jnp.int32),
            pltpu.VMEM((LOGCAP,), jnp.int32),
            pltpu.VMEM((LOGCAP,), jnp.int32),
            pltpu.VMEM((LOGCAP,), jnp.int32),
            pltpu.VMEM((LOGCAP,), jnp.int32),
            pltpu.VMEM((2 * GB, D), jnp.float32),
            pltpu.VMEM((SLAB, D), jnp.float32),
            pltpu.VMEM((16,), jnp.int32),
            pltpu.VMEM((16,), jnp.int32),
            pltpu.SemaphoreType.DMA,
            pltpu.SemaphoreType.DMA,
            pltpu.SemaphoreType.DMA,
            pltpu.SemaphoreType.DMA,
        ],
    )
    def sc_agg(x_hbm, src_hbm, dst_hbm, agg_hbm,
               srcbuf, dstbuf, logsrc0, logdst0, logsrc1, logdst1, rows, slab,
               sent_src, sent_dst, ls0, ls1, gs0, gs1):
        cid = lax.axis_index("c")
        sid = lax.axis_index("s")
        wid = sid * NC + cid
        base = wid * ROWS
        lsem = (ls0, ls1)
        gsem = (gs0, gs1)

        lanev = lax.iota(jnp.int32, 16)
        perms = [lanev ^ d for d in (1, 2, 4, 8)]

        sent_src[pl.ds(0, 16)] = jnp.broadcast_to(jnp.int32(SENT), (16,))
        sent_dst[pl.ds(0, 16)] = jnp.broadcast_to(jnp.int32(TRASH), (16,))

        zv = jnp.zeros((16,), jnp.float32)

        def zbody(r, carry):
            for j in range(D // 16):
                slab[r, pl.ds(j * 16, 16)] = zv
            return carry

        lax.fori_loop(0, SLAB, zbody, 0)

        def _ld_descs(p, sci):
            e0 = sci * SCE
            return (
                pltpu.make_async_copy(src_hbm.at[pl.ds(e0, SCE)],
                                      srcbuf.at[pl.ds(p * SCE, SCE)], lsem[p]),
                pltpu.make_async_copy(dst_hbm.at[pl.ds(e0, SCE)],
                                      dstbuf.at[pl.ds(p * SCE, SCE)], lsem[p]),
            )

        def _scan(p):
            lsrc = (logsrc0, logsrc1)[p]
            ldst = (logdst0, logdst1)[p]

            def chunk(k, cnt):
                vd = dstbuf[pl.ds(p * SCE + k * 16, 16)]
                vs = srcbuf[pl.ds(p * SCE + k * 16, 16)]
                u = vd - base
                inb = (u >= 0) & (u < ROWS)
                w0 = jnp.where(inb, u * 16 + lanev, BIG)
                n16 = _bsum(jnp.where(inb, 1, 0), perms)[0]

                def ext(i, carry2):
                    w, c = carry2
                    kminv = _bmin(w, perms)
                    lanemv = kminv & 15
                    lsrc[pl.ds(c, 16)] = _take16(vs, lanemv)
                    ldst[pl.ds(c, 16)] = kminv >> 4
                    w = jnp.where(lanev == lanemv, BIG, w)
                    return (w, c + 1)

                _, cnt = lax.fori_loop(0, n16, ext, (w0, cnt))
                return cnt

            return lax.fori_loop(0, SCE // 16, chunk, jnp.int32(0))

        def _acc16(q, lp):
            ldst = (logdst0, logdst1)[lp]
            dv = ldst[pl.ds(q * 16, 16)]
            for l in range(16):
                sl = dv[l]
                for j in range(D // 16):
                    slab[sl, pl.ds(j * 16, 16)] = (
                        slab[sl, pl.ds(j * 16, 16)]
                        + rows[((q & 7) * 16) + l, pl.ds(j * 16, 16)])

        def _d16(q, lp):
            lsrc = (logsrc0, logsrc1)[lp]
            return pltpu.make_async_copy(
                x_hbm.at[lsrc.at[pl.ds(q * 16, 16)]],
                rows.at[pl.ds((q & 7) * 16, 16)], gs0)

        def _issue(lp, lo, hi):
            def qs(q, carry):
                _d16(q, lp).start()
                return carry
            lax.fori_loop(lo, hi, qs, 0)

        def _drain_group(lp, lo, hi):
            def qw(q, carry):
                _d16(q, lp).wait()
                return carry
            lax.fori_loop(lo, hi, qw, 0)

            def qa(q, carry):
                _acc16(q, lp)
                return carry
            lax.fori_loop(lo, hi, qa, 0)

        def _finish(lp, nq):
            ngrp = (nq + 7) // 8

            def grp(g, carry):
                hi = jnp.minimum(nq, (g + 1) * 8)
                _drain_group(lp, g * 8, hi)
                hi2 = jnp.minimum(nq, (g + 2) * 8)
                _issue(lp, (g + 1) * 8, hi2)
                return carry

            lax.fori_loop(0, ngrp, grp, 0)

        def _pad(lp, cnt):
            lsrc = (logsrc0, logsrc1)[lp]
            ldst = (logdst0, logdst1)[lp]
            lsrc[pl.ds(cnt, 16)] = sent_src[pl.ds(0, 16)]
            ldst[pl.ds(cnt, 16)] = sent_dst[pl.ds(0, 16)]

        for d in _ld_descs(0, 0):
            d.start()

        def pairbody(pair, cnt_prev):
            for p in (0, 1):
                sci = pair * 2 + p
                pp = 1 - p

                for d in _ld_descs(p, sci):
                    d.wait()

                nxt = sci + 1

                @pl.when(nxt < NSC)
                def _():
                    for d in _ld_descs(1 - p, nxt):
                        d.start()

                nq_prev = (cnt_prev + 15) // 16
                _issue(pp, 0, jnp.minimum(nq_prev, 8))

                cnt = _scan(p)
                _pad(p, cnt)

                _finish(pp, nq_prev)
                cnt_prev = cnt
            return cnt_prev

        cnt_last = lax.fori_loop(0, NSC // 2, pairbody, jnp.int32(0))

        nq_last = (cnt_last + 15) // 16
        _issue(1, 0, jnp.minimum(nq_last, 8))
        _finish(1, nq_last)

        pltpu.sync_copy(slab.at[pl.ds(0, ROWS)],
                        agg_hbm.at[pl.ds(base, ROWS)])

    return sc_agg


_sc_agg = _sc_agg_build()


def _dense_body(agg_ref, x_ref, wr1_ref, wo1_ref, wr2_ref, wo2_ref,
                b1_ref, b2_ref, out_ref, score_ref):
    a = agg_ref[...]
    xb = x_ref[...]
    dn = (((1,), (1,)), ((), ()))
    s1 = (lax.dot_general(a, wr1_ref[...], dn,
                          preferred_element_type=jnp.float32)
          + lax.dot_general(xb, wo1_ref[...], dn,
                            preferred_element_type=jnp.float32)
          + b1_ref[0, 0])
    x1 = jnp.tanh(s1)
    x2 = (lax.dot_general(a, wr2_ref[...], dn,
                          preferred_element_type=jnp.float32)
          + lax.dot_general(xb, wo2_ref[...], dn,
                            preferred_element_type=jnp.float32)
          + b2_ref[...])
    g = x1 * x2
    out_ref[...] = jnp.where(g > 0, g, jnp.exp(jnp.minimum(g, 0.0)) - 1.0)
    score_ref[...] = x1


def _dense(x, agg, W_rel1, W_root1, W_rel2, W_root2, b1, b2):
    BN = 1000
    grid = (N_NODES // BN,)
    return pl.pallas_call(
        _dense_body,
        grid=grid,
        in_specs=[
            pl.BlockSpec((BN, D), lambda i: (i, 0)),
            pl.BlockSpec((BN, D), lambda i: (i, 0)),
            pl.BlockSpec((1, D), lambda i: (0, 0)),
            pl.BlockSpec((1, D), lambda i: (0, 0)),
            pl.BlockSpec((D, D), lambda i: (0, 0)),
            pl.BlockSpec((D, D), lambda i: (0, 0)),
            pl.BlockSpec((1, 1), lambda i: (0, 0)),
            pl.BlockSpec((1, D), lambda i: (0, 0)),
        ],
        out_specs=[
            pl.BlockSpec((BN, D), lambda i: (i, 0)),
            pl.BlockSpec((BN, 1), lambda i: (i, 0)),
        ],
        out_shape=[
            jax.ShapeDtypeStruct((N_NODES, D), jnp.float32),
            jax.ShapeDtypeStruct((N_NODES, 1), jnp.float32),
        ],
    )(agg, x, W_rel1, W_root1, W_rel2, W_root2, b1, b2)


def kernel(x, edge_index, W_rel1, b_rel1, W_root1, W_rel2, b_rel2, W_root2):
    src = edge_index[0].astype(jnp.int32)
    dst = edge_index[1].astype(jnp.int32)
    x_pad = jnp.concatenate([x, jnp.zeros((1, D), jnp.float32)], axis=0)
    agg = _sc_agg(x_pad, src, dst)[:N_NODES]
    b1 = b_rel1.reshape(1, 1).astype(jnp.float32)
    b2 = b_rel2.reshape(1, D).astype(jnp.float32)
    out, score = _dense(x, agg, W_rel1, W_root1, W_rel2, W_root2, b1, b2)
    return out, score.reshape(-1)


# speedup vs baseline: 1.2218x; 1.2218x over previous
import functools

import jax
import jax.numpy as jnp
import numpy as np
from jax import lax
from jax.experimental import pallas as pl
from jax.experimental.pallas import tpu as pltpu
from jax.experimental.pallas import tpu_sc as plsc

N_NODES = 10000
N_EDGES = 160000
D = 256

NC = 2
NS = 16
NW = NC * NS
ROWS = 320
TRASH = ROWS
SLAB = ROWS + 8
SENT = N_NODES
SCE = 1600
NSC = N_EDGES // SCE
GB = 64
LOGCAP = SCE + 32
BIG = np.int32(1 << 20)

_GDN = lax.GatherDimensionNumbers(offset_dims=(), collapsed_slice_dims=(0,),
                                  start_index_map=(0,))


def _take16(v, idx):
    return lax.gather(v, idx.reshape(16, 1), _GDN, (1,),
                      mode=lax.GatherScatterMode.PROMISE_IN_BOUNDS)


def _bmin(v, perms):
    for p in perms:
        v = jnp.minimum(v, _take16(v, p))
    return v


def _bsum(v, perms):
    for p in perms:
        v = v + _take16(v, p)
    return v


def _sc_agg_build():
    mesh = plsc.VectorSubcoreMesh(core_axis_name="c", subcore_axis_name="s")

    @functools.partial(
        pl.kernel,
        mesh=mesh,
        out_type=jax.ShapeDtypeStruct((NW * ROWS, D), jnp.float32),
        scratch_types=[
            pltpu.VMEM((2 * SCE,), jnp.int32),
            pltpu.VMEM((2 * SCE,), jnp.int32),
            pltpu.VMEM((LOGCAP,), jnp.int32),
            pltpu.VMEM((LOGCAP,), jnp.int32),
            pltpu.VMEM((LOGCAP,), jnp.int32),
            pltpu.VMEM((LOGCAP,), jnp.int32),
            pltpu.VMEM((2 * GB, D), jnp.float32),
            pltpu.VMEM((SLAB, D), jnp.float32),
            pltpu.VMEM((16,), jnp.int32),
            pltpu.VMEM((16,), jnp.int32),
            pltpu.SemaphoreType.DMA,
            pltpu.SemaphoreType.DMA,
            pltpu.SemaphoreType.DMA,
            pltpu.SemaphoreType.DMA,
        ],
    )
    def sc_agg(x_hbm, src_hbm, dst_hbm, agg_hbm,
               srcbuf, dstbuf, logsrc0, logdst0, logsrc1, logdst1, rows, slab,
               sent_src, sent_dst, ls0, ls1, gs0, gs1):
        cid = lax.axis_index("c")
        sid = lax.axis_index("s")
        wid = sid * NC + cid
        base = wid * ROWS
        lsem = (ls0, ls1)
        gsem = (gs0, gs1)

        lanev = lax.iota(jnp.int32, 16)
        perms = [lanev ^ d for d in (1, 2, 4, 8)]

        sent_src[pl.ds(0, 16)] = jnp.broadcast_to(jnp.int32(SENT), (16,))
        sent_dst[pl.ds(0, 16)] = jnp.broadcast_to(jnp.int32(TRASH), (16,))

        zv = jnp.zeros((16,), jnp.float32)

        def zbody(r, carry):
            for j in range(D // 16):
                slab[r, pl.ds(j * 16, 16)] = zv
            return carry

        lax.fori_loop(0, SLAB, zbody, 0)

        def _ld_descs(p, sci):
            e0 = sci * SCE
            return (
                pltpu.make_async_copy(src_hbm.at[pl.ds(e0, SCE)],
                                      srcbuf.at[pl.ds(p * SCE, SCE)], lsem[p]),
                pltpu.make_async_copy(dst_hbm.at[pl.ds(e0, SCE)],
                                      dstbuf.at[pl.ds(p * SCE, SCE)], lsem[p]),
            )

        def _scan(p):
            lsrc = (logsrc0, logsrc1)[p]
            ldst = (logdst0, logdst1)[p]

            def chunk(k, cnt):
                vd = dstbuf[pl.ds(p * SCE + k * 16, 16)]
                vs = srcbuf[pl.ds(p * SCE + k * 16, 16)]
                u = vd - base
                inb = (u >= 0) & (u < ROWS)
                w0 = jnp.where(inb, u * 16 + lanev, BIG)
                n16 = _bsum(jnp.where(inb, 1, 0), perms)[0]

                def ext(i, carry2):
                    w, c = carry2
                    kminv = _bmin(w, perms)
                    lanemv = kminv & 15
                    lsrc[pl.ds(c, 16)] = _take16(vs, lanemv)
                    ldst[pl.ds(c, 16)] = kminv >> 4
                    w = jnp.where(lanev == lanemv, BIG, w)
                    return (w, c + 1)

                _, cnt = lax.fori_loop(0, n16, ext, (w0, cnt))
                return cnt

            return lax.fori_loop(0, SCE // 16, chunk, jnp.int32(0))

        def _acc16(q, lp):
            ldst = (logdst0, logdst1)[lp]
            dv = ldst[pl.ds(q * 8, 16)]
            for l in range(8):
                sl = dv[l]
                for j in range(D // 16):
                    slab[sl, pl.ds(j * 16, 16)] = (
                        slab[sl, pl.ds(j * 16, 16)]
                        + rows[((q & 15) * 8) + l, pl.ds(j * 16, 16)])

        def _d16(q, lp):
            lsrc = (logsrc0, logsrc1)[lp]
            return pltpu.make_async_copy(
                x_hbm.at[lsrc.at[pl.ds(q * 8, 8)]],
                rows.at[pl.ds((q & 15) * 8, 8)], gs0)

        def _issue(lp, lo, hi):
            def qs(q, carry):
                _d16(q, lp).start()
                return carry
            lax.fori_loop(lo, hi, qs, 0)

        def _drain_group(lp, lo, hi):
            def qw(q, carry):
                _d16(q, lp).wait()
                return carry
            lax.fori_loop(lo, hi, qw, 0)

            def qa(q, carry):
                _acc16(q, lp)
                return carry
            lax.fori_loop(lo, hi, qa, 0)

        def _finish(lp, nq):
            ngrp = (nq + 15) // 16

            def grp(g, carry):
                hi = jnp.minimum(nq, (g + 1) * 16)
                _drain_group(lp, g * 16, hi)
                hi2 = jnp.minimum(nq, (g + 2) * 16)
                _issue(lp, (g + 1) * 16, hi2)
                return carry

            lax.fori_loop(0, ngrp, grp, 0)

        def _pad(lp, cnt):
            lsrc = (logsrc0, logsrc1)[lp]
            ldst = (logdst0, logdst1)[lp]
            lsrc[pl.ds(cnt, 16)] = sent_src[pl.ds(0, 16)]
            ldst[pl.ds(cnt, 16)] = sent_dst[pl.ds(0, 16)]

        for d in _ld_descs(0, 0):
            d.start()

        def pairbody(pair, cnt_prev):
            for p in (0, 1):
                sci = pair * 2 + p
                pp = 1 - p

                for d in _ld_descs(p, sci):
                    d.wait()

                nxt = sci + 1

                @pl.when(nxt < NSC)
                def _():
                    for d in _ld_descs(1 - p, nxt):
                        d.start()

                nq_prev = (cnt_prev + 7) // 8
                _issue(pp, 0, jnp.minimum(nq_prev, 16))

                cnt = _scan(p)
                _pad(p, cnt)

                _finish(pp, nq_prev)
                cnt_prev = cnt
            return cnt_prev

        cnt_last = lax.fori_loop(0, NSC // 2, pairbody, jnp.int32(0))

        nq_last = (cnt_last + 7) // 8
        _issue(1, 0, jnp.minimum(nq_last, 16))
        _finish(1, nq_last)

        pltpu.sync_copy(slab.at[pl.ds(0, ROWS)],
                        agg_hbm.at[pl.ds(base, ROWS)])

    return sc_agg


_sc_agg = _sc_agg_build()


def _dense_body(agg_ref, x_ref, wr1_ref, wo1_ref, wr2_ref, wo2_ref,
                b1_ref, b2_ref, out_ref, score_ref):
    a = agg_ref[...]
    xb = x_ref[...]
    dn = (((1,), (1,)), ((), ()))
    s1 = (lax.dot_general(a, wr1_ref[...], dn,
                          preferred_element_type=jnp.float32)
          + lax.dot_general(xb, wo1_ref[...], dn,
                            preferred_element_type=jnp.float32)
          + b1_ref[0, 0])
    x1 = jnp.tanh(s1)
    x2 = (lax.dot_general(a, wr2_ref[...], dn,
                          preferred_element_type=jnp.float32)
          + lax.dot_general(xb, wo2_ref[...], dn,
                            preferred_element_type=jnp.float32)
          + b2_ref[...])
    g = x1 * x2
    out_ref[...] = jnp.where(g > 0, g, jnp.exp(jnp.minimum(g, 0.0)) - 1.0)
    score_ref[...] = x1


def _dense(x, agg, W_rel1, W_root1, W_rel2, W_root2, b1, b2):
    BN = 1000
    grid = (N_NODES // BN,)
    return pl.pallas_call(
        _dense_body,
        grid=grid,
        in_specs=[
            pl.BlockSpec((BN, D), lambda i: (i, 0)),
            pl.BlockSpec((BN, D), lambda i: (i, 0)),
            pl.BlockSpec((1, D), lambda i: (0, 0)),
            pl.BlockSpec((1, D), lambda i: (0, 0)),
            pl.BlockSpec((D, D), lambda i: (0, 0)),
            pl.BlockSpec((D, D), lambda i: (0, 0)),
            pl.BlockSpec((1, 1), lambda i: (0, 0)),
            pl.BlockSpec((1, D), lambda i: (0, 0)),
        ],
        out_specs=[
            pl.BlockSpec((BN, D), lambda i: (i, 0)),
            pl.BlockSpec((BN, 1), lambda i: (i, 0)),
        ],
        out_shape=[
            jax.ShapeDtypeStruct((N_NODES, D), jnp.float32),
            jax.ShapeDtypeStruct((N_NODES, 1), jnp.float32),
        ],
    )(agg, x, W_rel1, W_root1, W_rel2, W_root2, b1, b2)


def kernel(x, edge_index, W_rel1, b_rel1, W_root1, W_rel2, b_rel2, W_root2):
    src = edge_index[0].astype(jnp.int32)
    dst = edge_index[1].astype(jnp.int32)
    x_pad = jnp.concatenate([x, jnp.zeros((1, D), jnp.float32)], axis=0)
    agg = _sc_agg(x_pad, src, dst)[:N_NODES]
    b1 = b_rel1.reshape(1, 1).astype(jnp.float32)
    b2 = b_rel2.reshape(1, D).astype(jnp.float32)
    out, score = _dense(x, agg, W_rel1, W_root1, W_rel2, W_root2, b1, b2)
    return out, score.reshape(-1)


# speedup vs baseline: 1.4433x; 1.1813x over previous
import functools

import jax
import jax.numpy as jnp
import numpy as np
from jax import lax
from jax.experimental import pallas as pl
from jax.experimental.pallas import tpu as pltpu
from jax.experimental.pallas import tpu_sc as plsc

N_NODES = 10000
N_EDGES = 160000
D = 256

NC = 2
NS = 16
NW = NC * NS
ROWS = 320
TRASH = ROWS
SLAB = ROWS + 8
SENT = N_NODES
SCE = 1600
NSC = N_EDGES // SCE
GB = 64
LOGCAP = SCE + 32
BIG = np.int32(1 << 20)

_GDN = lax.GatherDimensionNumbers(offset_dims=(), collapsed_slice_dims=(0,),
                                  start_index_map=(0,))


def _take16(v, idx):
    return lax.gather(v, idx.reshape(16, 1), _GDN, (1,),
                      mode=lax.GatherScatterMode.PROMISE_IN_BOUNDS)


def _bmin(v, perms):
    for p in perms:
        v = jnp.minimum(v, _take16(v, p))
    return v


def _bsum(v, perms):
    for p in perms:
        v = v + _take16(v, p)
    return v


def _sc_agg_build():
    mesh = plsc.VectorSubcoreMesh(core_axis_name="c", subcore_axis_name="s")

    @functools.partial(
        pl.kernel,
        mesh=mesh,
        out_type=jax.ShapeDtypeStruct((NW * ROWS, D), jnp.float32),
        scratch_types=[
            pltpu.VMEM((2 * SCE,), jnp.int32),
            pltpu.VMEM((2 * SCE,), jnp.int32),
            pltpu.VMEM((LOGCAP,), jnp.int32),
            pltpu.VMEM((LOGCAP,), jnp.int32),
            pltpu.VMEM((LOGCAP,), jnp.int32),
            pltpu.VMEM((LOGCAP,), jnp.int32),
            pltpu.VMEM((2 * GB, D), jnp.float32),
            pltpu.VMEM((SLAB, D), jnp.float32),
            pltpu.VMEM((16,), jnp.int32),
            pltpu.VMEM((16,), jnp.int32),
            pltpu.SemaphoreType.DMA,
            pltpu.SemaphoreType.DMA,
            pltpu.SemaphoreType.DMA,
            pltpu.SemaphoreType.DMA,
        ],
    )
    def sc_agg(x_hbm, src_hbm, dst_hbm, agg_hbm,
               srcbuf, dstbuf, logsrc0, logdst0, logsrc1, logdst1, rows, slab,
               sent_src, sent_dst, ls0, ls1, gs0, gs1):
        cid = lax.axis_index("c")
        sid = lax.axis_index("s")
        wid = sid * NC + cid
        base = wid * ROWS
        lsem = (ls0, ls1)
        gsem = (gs0, gs1)

        lanev = lax.iota(jnp.int32, 16)
        perms = [lanev ^ d for d in (1, 2, 4, 8)]

        sent_src[pl.ds(0, 16)] = jnp.broadcast_to(jnp.int32(SENT), (16,))
        sent_dst[pl.ds(0, 16)] = jnp.broadcast_to(jnp.int32(TRASH), (16,))

        zv = jnp.zeros((16,), jnp.float32)

        def zbody(r, carry):
            for j in range(D // 16):
                slab[r, pl.ds(j * 16, 16)] = zv
            return carry

        lax.fori_loop(0, SLAB, zbody, 0)

        def _ld_descs(p, sci):
            e0 = sci * SCE
            return (
                pltpu.make_async_copy(src_hbm.at[pl.ds(e0, SCE)],
                                      srcbuf.at[pl.ds(p * SCE, SCE)], lsem[p]),
                pltpu.make_async_copy(dst_hbm.at[pl.ds(e0, SCE)],
                                      dstbuf.at[pl.ds(p * SCE, SCE)], lsem[p]),
            )

        def _scan(p):
            lsrc = (logsrc0, logsrc1)[p]
            ldst = (logdst0, logdst1)[p]

            def chunk(k, cnt):
                vd_a = dstbuf[pl.ds(p * SCE + k * 32, 16)]
                vs_a = srcbuf[pl.ds(p * SCE + k * 32, 16)]
                vd_b = dstbuf[pl.ds(p * SCE + k * 32 + 16, 16)]
                vs_b = srcbuf[pl.ds(p * SCE + k * 32 + 16, 16)]
                u_a = vd_a - base
                u_b = vd_b - base
                inb_a = (u_a >= 0) & (u_a < ROWS)
                inb_b = (u_b >= 0) & (u_b < ROWS)
                w0_a = jnp.where(inb_a, u_a * 16 + lanev, BIG)
                w0_b = jnp.where(inb_b, u_b * 16 + lanev, BIG)
                na = _bsum(jnp.where(inb_a, 1, 0), perms)[0]
                nb_ = _bsum(jnp.where(inb_b, 1, 0), perms)[0]

                def mkext(vs):
                    def ext(i, carry2):
                        w, c = carry2
                        kminv = _bmin(w, perms)
                        lanemv = kminv & 15
                        lsrc[pl.ds(c, 16)] = _take16(vs, lanemv)
                        ldst[pl.ds(c, 16)] = kminv >> 4
                        w = jnp.where(lanev == lanemv, BIG, w)
                        return (w, c + 1)
                    return ext

                _, cnt = lax.fori_loop(0, na, mkext(vs_a), (w0_a, cnt))
                _, cnt = lax.fori_loop(0, nb_, mkext(vs_b), (w0_b, cnt))
                return cnt

            return lax.fori_loop(0, SCE // 32, chunk, jnp.int32(0))

        def _acc16(q, lp):
            ldst = (logdst0, logdst1)[lp]
            dv = ldst[pl.ds(q * 8, 16)]
            for l in range(8):
                sl = dv[l]
                for j in range(D // 16):
                    slab[sl, pl.ds(j * 16, 16)] = (
                        slab[sl, pl.ds(j * 16, 16)]
                        + rows[((q & 15) * 8) + l, pl.ds(j * 16, 16)])

        def _d16(q, lp):
            lsrc = (logsrc0, logsrc1)[lp]
            return pltpu.make_async_copy(
                x_hbm.at[lsrc.at[pl.ds(q * 8, 8)]],
                rows.at[pl.ds((q & 15) * 8, 8)], gs0)

        def _issue(lp, lo, hi):
            def qs(q, carry):
                _d16(q, lp).start()
                return carry
            lax.fori_loop(lo, hi, qs, 0)

        def _drain_group(lp, lo, hi):
            def qw(q, carry):
                _d16(q, lp).wait()
                return carry
            lax.fori_loop(lo, hi, qw, 0)

            def qa(q, carry):
                _acc16(q, lp)
                return carry
            lax.fori_loop(lo, hi, qa, 0)

        def _finish(lp, nq):
            ngrp = (nq + 15) // 16

            def grp(g, carry):
                hi = jnp.minimum(nq, (g + 1) * 16)
                _drain_group(lp, g * 16, hi)
                hi2 = jnp.minimum(nq, (g + 2) * 16)
                _issue(lp, (g + 1) * 16, hi2)
                return carry

            lax.fori_loop(0, ngrp, grp, 0)

        def _pad(lp, cnt):
            lsrc = (logsrc0, logsrc1)[lp]
            ldst = (logdst0, logdst1)[lp]
            lsrc[pl.ds(cnt, 16)] = sent_src[pl.ds(0, 16)]
            ldst[pl.ds(cnt, 16)] = sent_dst[pl.ds(0, 16)]

        for d in _ld_descs(0, 0):
            d.start()

        def pairbody(pair, cnt_prev):
            for p in (0, 1):
                sci = pair * 2 + p
                pp = 1 - p

                for d in _ld_descs(p, sci):
                    d.wait()

                nxt = sci + 1

                @pl.when(nxt < NSC)
                def _():
                    for d in _ld_descs(1 - p, nxt):
                        d.start()

                nq_prev = (cnt_prev + 7) // 8
                _issue(pp, 0, jnp.minimum(nq_prev, 16))

                cnt = _scan(p)
                _pad(p, cnt)

                _finish(pp, nq_prev)
                cnt_prev = cnt
            return cnt_prev

        cnt_last = lax.fori_loop(0, NSC // 2, pairbody, jnp.int32(0))

        nq_last = (cnt_last + 7) // 8
        _issue(1, 0, jnp.minimum(nq_last, 16))
        _finish(1, nq_last)

        pltpu.sync_copy(slab.at[pl.ds(0, ROWS)],
                        agg_hbm.at[pl.ds(base, ROWS)])

    return sc_agg


_sc_agg = _sc_agg_build()


def _dense_body(agg_ref, x_ref, wr1_ref, wo1_ref, wr2_ref, wo2_ref,
                b1_ref, b2_ref, out_ref, score_ref):
    a = agg_ref[...]
    xb = x_ref[...]
    dn = (((1,), (1,)), ((), ()))
    s1 = (lax.dot_general(a, wr1_ref[...], dn,
                          preferred_element_type=jnp.float32)
          + lax.dot_general(xb, wo1_ref[...], dn,
                            preferred_element_type=jnp.float32)
          + b1_ref[0, 0])
    x1 = jnp.tanh(s1)
    x2 = (lax.dot_general(a, wr2_ref[...], dn,
                          preferred_element_type=jnp.float32)
          + lax.dot_general(xb, wo2_ref[...], dn,
                            preferred_element_type=jnp.float32)
          + b2_ref[...])
    g = x1 * x2
    out_ref[...] = jnp.where(g > 0, g, jnp.exp(jnp.minimum(g, 0.0)) - 1.0)
    score_ref[...] = x1


def _dense(x, agg, W_rel1, W_root1, W_rel2, W_root2, b1, b2):
    BN = 1000
    grid = (N_NODES // BN,)
    return pl.pallas_call(
        _dense_body,
        grid=grid,
        in_specs=[
            pl.BlockSpec((BN, D), lambda i: (i, 0)),
            pl.BlockSpec((BN, D), lambda i: (i, 0)),
            pl.BlockSpec((1, D), lambda i: (0, 0)),
            pl.BlockSpec((1, D), lambda i: (0, 0)),
            pl.BlockSpec((D, D), lambda i: (0, 0)),
            pl.BlockSpec((D, D), lambda i: (0, 0)),
            pl.BlockSpec((1, 1), lambda i: (0, 0)),
            pl.BlockSpec((1, D), lambda i: (0, 0)),
        ],
        out_specs=[
            pl.BlockSpec((BN, D), lambda i: (i, 0)),
            pl.BlockSpec((BN, 1), lambda i: (i, 0)),
        ],
        out_shape=[
            jax.ShapeDtypeStruct((N_NODES, D), jnp.float32),
            jax.ShapeDtypeStruct((N_NODES, 1), jnp.float32),
        ],
    )(agg, x, W_rel1, W_root1, W_rel2, W_root2, b1, b2)


def kernel(x, edge_index, W_rel1, b_rel1, W_root1, W_rel2, b_rel2, W_root2):
    src = edge_index[0].astype(jnp.int32)
    dst = edge_index[1].astype(jnp.int32)
    x_pad = jnp.concatenate([x, jnp.zeros((1, D), jnp.float32)], axis=0)
    agg = _sc_agg(x_pad, src, dst)[:N_NODES]
    b1 = b_rel1.reshape(1, 1).astype(jnp.float32)
    b2 = b_rel2.reshape(1, D).astype(jnp.float32)
    out, score = _dense(x, agg, W_rel1, W_root1, W_rel2, W_root2, b1, b2)
    return out, score.reshape(-1)


# speedup vs baseline: 1.5138x; 1.0488x over previous
import functools

import jax
import jax.numpy as jnp
import numpy as np
from jax import lax
from jax.experimental import pallas as pl
from jax.experimental.pallas import tpu as pltpu
from jax.experimental.pallas import tpu_sc as plsc

N_NODES = 10000
N_EDGES = 160000
D = 256

NC = 2
NS = 16
NW = NC * NS
ROWS = 320
TRASH = ROWS
SLAB = ROWS + 8
SENT = N_NODES
SCE = 1600
NSC = N_EDGES // SCE
GB = 64
LOGCAP = SCE + 32
BIG = np.int32(1 << 20)

_GDN = lax.GatherDimensionNumbers(offset_dims=(), collapsed_slice_dims=(0,),
                                  start_index_map=(0,))


def _take16(v, idx):
    return lax.gather(v, idx.reshape(16, 1), _GDN, (1,),
                      mode=lax.GatherScatterMode.PROMISE_IN_BOUNDS)


def _bmin(v, perms):
    for p in perms:
        v = jnp.minimum(v, _take16(v, p))
    return v


def _bsum(v, perms):
    for p in perms:
        v = v + _take16(v, p)
    return v


def _sc_agg_build():
    mesh = plsc.VectorSubcoreMesh(core_axis_name="c", subcore_axis_name="s")

    @functools.partial(
        pl.kernel,
        mesh=mesh,
        out_type=jax.ShapeDtypeStruct((NW * ROWS, D), jnp.float32),
        scratch_types=[
            pltpu.VMEM((2 * SCE,), jnp.int32),
            pltpu.VMEM((2 * SCE,), jnp.int32),
            pltpu.VMEM((LOGCAP,), jnp.int32),
            pltpu.VMEM((LOGCAP,), jnp.int32),
            pltpu.VMEM((LOGCAP,), jnp.int32),
            pltpu.VMEM((LOGCAP,), jnp.int32),
            pltpu.VMEM((2 * GB, D), jnp.float32),
            pltpu.VMEM((SLAB, D), jnp.float32),
            pltpu.VMEM((16,), jnp.int32),
            pltpu.VMEM((16,), jnp.int32),
            pltpu.SemaphoreType.DMA,
            pltpu.SemaphoreType.DMA,
            pltpu.SemaphoreType.DMA,
            pltpu.SemaphoreType.DMA,
        ],
    )
    def sc_agg(x_hbm, src_hbm, dst_hbm, agg_hbm,
               srcbuf, dstbuf, logsrc0, logdst0, logsrc1, logdst1, rows, slab,
               sent_src, sent_dst, ls0, ls1, gs0, gs1):
        cid = lax.axis_index("c")
        sid = lax.axis_index("s")
        wid = sid * NC + cid
        base = wid * ROWS
        lsem = (ls0, ls1)
        gsem = (gs0, gs1)

        lanev = lax.iota(jnp.int32, 16)
        perms = [lanev ^ d for d in (1, 2, 4, 8)]

        sent_src[pl.ds(0, 16)] = jnp.broadcast_to(jnp.int32(SENT), (16,))
        sent_dst[pl.ds(0, 16)] = jnp.broadcast_to(jnp.int32(TRASH), (16,))

        zv = jnp.zeros((16,), jnp.float32)

        def zbody(r, carry):
            for j in range(D // 16):
                slab[r, pl.ds(j * 16, 16)] = zv
            return carry

        lax.fori_loop(0, SLAB, zbody, 0)

        def _ld_descs(p, sci):
            e0 = sci * SCE
            return (
                pltpu.make_async_copy(src_hbm.at[pl.ds(e0, SCE)],
                                      srcbuf.at[pl.ds(p * SCE, SCE)], lsem[p]),
                pltpu.make_async_copy(dst_hbm.at[pl.ds(e0, SCE)],
                                      dstbuf.at[pl.ds(p * SCE, SCE)], lsem[p]),
            )

        def _scan(p):
            lsrc = (logsrc0, logsrc1)[p]
            ldst = (logdst0, logdst1)[p]

            def chunk(k, cnt):
                vds, vss, w0s, ns = [], [], [], []
                for t in range(4):
                    vd = dstbuf[pl.ds(p * SCE + k * 64 + t * 16, 16)]
                    vs = srcbuf[pl.ds(p * SCE + k * 64 + t * 16, 16)]
                    u = vd - base
                    inb = (u >= 0) & (u < ROWS)
                    w0s.append(jnp.where(inb, u * 16 + lanev, BIG))
                    ns.append(_bsum(jnp.where(inb, 1, 0), perms)[0])
                    vss.append(vs)

                def mkext(vs):
                    def ext(i, carry2):
                        w, c = carry2
                        kminv = _bmin(w, perms)
                        lanemv = kminv & 15
                        lsrc[pl.ds(c, 16)] = _take16(vs, lanemv)
                        ldst[pl.ds(c, 16)] = kminv >> 4
                        w = jnp.where(lanev == lanemv, BIG, w)
                        return (w, c + 1)
                    return ext

                for t in range(4):
                    _, cnt = lax.fori_loop(0, ns[t], mkext(vss[t]),
                                           (w0s[t], cnt))
                return cnt

            return lax.fori_loop(0, SCE // 64, chunk, jnp.int32(0))

        def _acc16(q, lp):
            ldst = (logdst0, logdst1)[lp]
            dv = ldst[pl.ds(q * 8, 16)]
            for l in range(8):
                sl = dv[l]
                for j in range(D // 16):
                    slab[sl, pl.ds(j * 16, 16)] = (
                        slab[sl, pl.ds(j * 16, 16)]
                        + rows[((q & 15) * 8) + l, pl.ds(j * 16, 16)])

        def _d16(q, lp):
            lsrc = (logsrc0, logsrc1)[lp]
            return pltpu.make_async_copy(
                x_hbm.at[lsrc.at[pl.ds(q * 8, 8)]],
                rows.at[pl.ds((q & 15) * 8, 8)], gs0)

        def _issue(lp, lo, hi):
            def qs(q, carry):
                _d16(q, lp).start()
                return carry
            lax.fori_loop(lo, hi, qs, 0)

        def _drain_group(lp, lo, hi):
            def qw(q, carry):
                _d16(q, lp).wait()
                return carry
            lax.fori_loop(lo, hi, qw, 0)

            def qa(q, carry):
                _acc16(q, lp)
                return carry
            lax.fori_loop(lo, hi, qa, 0)

        def _finish(lp, nq):
            ngrp = (nq + 15) // 16

            def grp(g, carry):
                hi = jnp.minimum(nq, (g + 1) * 16)
                _drain_group(lp, g * 16, hi)
                hi2 = jnp.minimum(nq, (g + 2) * 16)
                _issue(lp, (g + 1) * 16, hi2)
                return carry

            lax.fori_loop(0, ngrp, grp, 0)

        def _pad(lp, cnt):
            lsrc = (logsrc0, logsrc1)[lp]
            ldst = (logdst0, logdst1)[lp]
            lsrc[pl.ds(cnt, 16)] = sent_src[pl.ds(0, 16)]
            ldst[pl.ds(cnt, 16)] = sent_dst[pl.ds(0, 16)]

        for d in _ld_descs(0, 0):
            d.start()

        def pairbody(pair, cnt_prev):
            for p in (0, 1):
                sci = pair * 2 + p
                pp = 1 - p

                for d in _ld_descs(p, sci):
                    d.wait()

                nxt = sci + 1

                @pl.when(nxt < NSC)
                def _():
                    for d in _ld_descs(1 - p, nxt):
                        d.start()

                nq_prev = (cnt_prev + 7) // 8
                _issue(pp, 0, jnp.minimum(nq_prev, 16))

                cnt = _scan(p)
                _pad(p, cnt)

                _finish(pp, nq_prev)
                cnt_prev = cnt
            return cnt_prev

        cnt_last = lax.fori_loop(0, NSC // 2, pairbody, jnp.int32(0))

        nq_last = (cnt_last + 7) // 8
        _issue(1, 0, jnp.minimum(nq_last, 16))
        _finish(1, nq_last)

        pltpu.sync_copy(slab.at[pl.ds(0, ROWS)],
                        agg_hbm.at[pl.ds(base, ROWS)])

    return sc_agg


_sc_agg = _sc_agg_build()


def _dense_body(agg_ref, x_ref, wr1_ref, wo1_ref, wr2_ref, wo2_ref,
                b1_ref, b2_ref, out_ref, score_ref):
    a = agg_ref[...]
    xb = x_ref[...]
    dn = (((1,), (1,)), ((), ()))
    s1 = (lax.dot_general(a, wr1_ref[...], dn,
                          preferred_element_type=jnp.float32)
          + lax.dot_general(xb, wo1_ref[...], dn,
                            preferred_element_type=jnp.float32)
          + b1_ref[0, 0])
    x1 = jnp.tanh(s1)
    x2 = (lax.dot_general(a, wr2_ref[...], dn,
                          preferred_element_type=jnp.float32)
          + lax.dot_general(xb, wo2_ref[...], dn,
                            preferred_element_type=jnp.float32)
          + b2_ref[...])
    g = x1 * x2
    out_ref[...] = jnp.where(g > 0, g, jnp.exp(jnp.minimum(g, 0.0)) - 1.0)
    score_ref[...] = x1


def _dense(x, agg, W_rel1, W_root1, W_rel2, W_root2, b1, b2):
    BN = 1000
    grid = (N_NODES // BN,)
    return pl.pallas_call(
        _dense_body,
        grid=grid,
        in_specs=[
            pl.BlockSpec((BN, D), lambda i: (i, 0)),
            pl.BlockSpec((BN, D), lambda i: (i, 0)),
            pl.BlockSpec((1, D), lambda i: (0, 0)),
            pl.BlockSpec((1, D), lambda i: (0, 0)),
            pl.BlockSpec((D, D), lambda i: (0, 0)),
            pl.BlockSpec((D, D), lambda i: (0, 0)),
            pl.BlockSpec((1, 1), lambda i: (0, 0)),
            pl.BlockSpec((1, D), lambda i: (0, 0)),
        ],
        out_specs=[
            pl.BlockSpec((BN, D), lambda i: (i, 0)),
            pl.BlockSpec((BN, 1), lambda i: (i, 0)),
        ],
        out_shape=[
            jax.ShapeDtypeStruct((N_NODES, D), jnp.float32),
            jax.ShapeDtypeStruct((N_NODES, 1), jnp.float32),
        ],
    )(agg, x, W_rel1, W_root1, W_rel2, W_root2, b1, b2)


def kernel(x, edge_index, W_rel1, b_rel1, W_root1, W_rel2, b_rel2, W_root2):
    src = edge_index[0].astype(jnp.int32)
    dst = edge_index[1].astype(jnp.int32)
    x_pad = jnp.concatenate([x, jnp.zeros((1, D), jnp.float32)], axis=0)
    agg = _sc_agg(x_pad, src, dst)[:N_NODES]
    b1 = b_rel1.reshape(1, 1).astype(jnp.float32)
    b2 = b_rel2.reshape(1, D).astype(jnp.float32)
    out, score = _dense(x, agg, W_rel1, W_root1, W_rel2, W_root2, b1, b2)
    return out, score.reshape(-1)
